# baseline (speedup 1.0000x reference)
import functools
import math

import jax
import jax.numpy as jnp
from jax.experimental import pallas as pl
from jax.experimental.pallas import tpu as pltpu

_NUM_CONN = 16
_R2 = 1000.0 * 1000.0
_F32 = jnp.float32


def _pad2(a, rows, cols):
    return jnp.pad(a, ((0, rows - a.shape[0]), (0, cols - a.shape[1])))


def _mlp_val(h, wbs, plain_last=False):
    n = len(wbs)
    for i, (w_ref, b_ref) in enumerate(wbs):
        h = jnp.dot(h, w_ref[...], preferred_element_type=_F32) + b_ref[...]
        if not (plain_last and i == n - 1):
            h = jnp.maximum(h, 0.0)
    return h


def _fps_kernel(px_ref, py_ref, pz_ref, idx_ref, qx_ref, qy_ref, qz_ref,
                *, n_real, m):
    px = px_ref[...]
    py = py_ref[...]
    pz = pz_ref[...]
    shp = px.shape
    gidx = (jax.lax.broadcasted_iota(jnp.int32, shp, 0) * shp[1]
            + jax.lax.broadcasted_iota(jnp.int32, shp, 1))
    dists0 = jnp.where(gidx < n_real, jnp.inf, -1.0).astype(_F32)

    idx_ref[pl.ds(0, 1), :] = jnp.zeros((1, 1), jnp.int32)
    l0x = px[0, 0]
    l0y = py[0, 0]
    l0z = pz[0, 0]
    qx_ref[pl.ds(0, 1), :] = jnp.reshape(l0x, (1, 1))
    qy_ref[pl.ds(0, 1), :] = jnp.reshape(l0y, (1, 1))
    qz_ref[pl.ds(0, 1), :] = jnp.reshape(l0z, (1, 1))

    def body(i, carry):
        dists, lx, ly, lz = carry
        dx = px - lx
        dy = py - ly
        dz = pz - lz
        d = dx * dx + dy * dy + dz * dz
        dists = jnp.minimum(dists, d)
        mx = jnp.max(dists)
        j = jnp.min(jnp.where(dists == mx, gidx, jnp.int32(1 << 30)))
        onehot = gidx == j
        nlx = jnp.sum(jnp.where(onehot, px, 0.0))
        nly = jnp.sum(jnp.where(onehot, py, 0.0))
        nlz = jnp.sum(jnp.where(onehot, pz, 0.0))
        idx_ref[pl.ds(i, 1), :] = jnp.reshape(j, (1, 1))
        qx_ref[pl.ds(i, 1), :] = jnp.reshape(nlx, (1, 1))
        qy_ref[pl.ds(i, 1), :] = jnp.reshape(nly, (1, 1))
        qz_ref[pl.ds(i, 1), :] = jnp.reshape(nlz, (1, 1))
        return dists, nlx, nly, nlz

    jax.lax.fori_loop(1, m, body, (dists0, l0x, l0y, l0z))


def _fps(pos, n_real, m):
    n_pad = pos.shape[0]
    rows = n_pad // 128
    px = pos[:, 0].reshape(rows, 128)
    py = pos[:, 1].reshape(rows, 128)
    pz = pos[:, 2].reshape(rows, 128)
    m_pad = ((m + 7) // 8) * 8
    out = pl.pallas_call(
        functools.partial(_fps_kernel, n_real=n_real, m=m),
        out_shape=[
            jax.ShapeDtypeStruct((m_pad, 1), jnp.int32),
            jax.ShapeDtypeStruct((m_pad, 1), _F32),
            jax.ShapeDtypeStruct((m_pad, 1), _F32),
            jax.ShapeDtypeStruct((m_pad, 1), _F32),
        ],
    )(px, py, pz)
    _, qx, qy, qz = out
    return jnp.concatenate([qx, qy, qz], axis=1)[:m]


def _sa_kernel(posq_ref, post_ref, g_ref, w1, b1, w2, b2, w3, b3,
               out_ref, e_scr, v_scr, *, n_cand, c_feat, k_nn):
    posq = posq_ref[...]
    post = post_ref[...]
    bq = posq.shape[0]
    np_ = post.shape[1]
    aa = jnp.sum(posq * posq, axis=1, keepdims=True)
    bb = jnp.sum(post * post, axis=0, keepdims=True)
    ab = jnp.dot(posq, post, preferred_element_type=_F32)
    d2 = jnp.maximum(aa + bb - 2.0 * ab, 0.0)
    col = jax.lax.broadcasted_iota(jnp.int32, (bq, np_), 1)
    d2 = jnp.where((col < n_cand) & (d2 <= _R2), d2, jnp.inf)
    g = g_ref[...]

    for k in range(k_nn):
        mn = jnp.min(d2, axis=1, keepdims=True)
        sel = jnp.min(jnp.where(d2 == mn, col, jnp.int32(1 << 30)),
                      axis=1, keepdims=True)
        onehot = col == sel
        d2 = jnp.where(onehot, jnp.inf, d2)
        e = jnp.dot(onehot.astype(_F32), g, preferred_element_type=_F32)
        e_scr[pl.ds(k * bq, bq), :] = e
        v_scr[pl.ds(k * bq, bq), :] = (mn < jnp.inf).astype(_F32)

    cg = g.shape[1]
    e_all = e_scr[...]
    posq3 = posq[:, :3]
    corr = jnp.concatenate(
        [jnp.zeros((k_nn * bq, c_feat), _F32),
         jnp.tile(posq3, (k_nn, 1)),
         jnp.zeros((k_nn * bq, cg - c_feat - 3), _F32)], axis=1)
    h = _mlp_val(e_all - corr, [(w1, b1), (w2, b2), (w3, b3)])
    h = jnp.where(v_scr[...] > 0.0, h, -jnp.inf)
    h = jnp.max(jnp.reshape(h, (k_nn, bq, h.shape[1])), axis=0)
    out_ref[...] = h


def _sa_conv(posq, pos_cand, feats, nn, n_cand, block_q):
    mq_pad = posq.shape[0]
    block_q = min(block_q, mq_pad)
    np_pad = pos_cand.shape[0]
    c_feat = feats.shape[1]
    cg = ((c_feat + 3 + 7) // 8) * 8
    g = _pad2(jnp.concatenate([feats, pos_cand], axis=1), np_pad, cg)
    posq8 = _pad2(posq, mq_pad, 8)
    post = _pad2(pos_cand.T, 8, np_pad)
    (w1, bb1), (w2, bb2), (w3, bb3) = nn
    w1p = _pad2(w1, cg, w1.shape[1])
    h3 = w3.shape[1]
    grid = mq_pad // block_q
    kern = functools.partial(_sa_kernel, n_cand=n_cand, c_feat=c_feat,
                             k_nn=_NUM_CONN)
    full = lambda s: pl.BlockSpec(s, lambda i: (0, 0))
    return pl.pallas_call(
        kern,
        grid=(grid,),
        in_specs=[
            pl.BlockSpec((block_q, 8), lambda i: (i, 0)),
            full((8, np_pad)),
            full((np_pad, cg)),
            full(w1p.shape), full((1, w1.shape[1])),
            full(w2.shape), full((1, w2.shape[1])),
            full(w3.shape), full((1, h3)),
        ],
        out_specs=pl.BlockSpec((block_q, h3), lambda i: (i, 0)),
        out_shape=jax.ShapeDtypeStruct((mq_pad, h3), _F32),
        scratch_shapes=[
            pltpu.VMEM((_NUM_CONN * block_q, cg), _F32),
            pltpu.VMEM((_NUM_CONN * block_q, 1), _F32),
        ],
    )(posq8, post, g, w1p, bb1.reshape(1, -1), w2, bb2.reshape(1, -1),
      w3, bb3.reshape(1, -1))


def _gsa_kernel(x2_ref, pos2_ref, gw1, gb1, gw2, gb2, gw3, gb3,
                fw1, fb1, fw2, fb2, fw3, fb3, out_ref, *, n_real):
    x2 = x2_ref[...]
    pos2 = pos2_ref[...][:, :3]
    h = jnp.concatenate([x2, pos2], axis=1)
    hg = _mlp_val(h, [(gw1, gb1), (gw2, gb2), (gw3, gb3)])
    row = jax.lax.broadcasted_iota(jnp.int32, hg.shape, 0)
    hg = jnp.where(row < n_real, hg, -jnp.inf)
    xg = jnp.max(hg, axis=0, keepdims=True)
    xg_b = jnp.broadcast_to(xg, (x2.shape[0], xg.shape[1]))
    fin = jnp.concatenate([xg_b, x2], axis=1)
    out_ref[...] = _mlp_val(fin, [(fw1, fb1), (fw2, fb2), (fw3, fb3)])


def _gsa_fp1(x2, pos2, gsa_nn, fp1_nn, n_real):
    rows = x2.shape[0]
    args = [x2, _pad2(pos2, rows, 8)]
    for (w, b) in list(gsa_nn) + list(fp1_nn):
        args += [w, b.reshape(1, -1)]
    h_out = fp1_nn[-1][0].shape[1]
    return pl.pallas_call(
        functools.partial(_gsa_kernel, n_real=n_real),
        out_shape=jax.ShapeDtypeStruct((rows, h_out), _F32),
    )(*args)


def _fp_kernel(posq_ref, post_ref, hsrc_ref, skip_ref,
               *mlp_refs, n_src, k_nn, plain_last):
    out_ref = mlp_refs[-1]
    wbs = [(mlp_refs[2 * i], mlp_refs[2 * i + 1])
           for i in range((len(mlp_refs) - 1) // 2)]
    posq = posq_ref[...]
    post = post_ref[...]
    bq = posq.shape[0]
    np_ = post.shape[1]
    aa = jnp.sum(posq * posq, axis=1, keepdims=True)
    bb = jnp.sum(post * post, axis=0, keepdims=True)
    ab = jnp.dot(posq, post, preferred_element_type=_F32)
    d2 = jnp.maximum(aa + bb - 2.0 * ab, 0.0)
    col = jax.lax.broadcasted_iota(jnp.int32, (bq, np_), 1)
    d2 = jnp.where(col < n_src, d2, jnp.inf)

    wmat = jnp.zeros((bq, np_), _F32)
    wsum = jnp.zeros((bq, 1), _F32)
    for _ in range(k_nn):
        mn = jnp.min(d2, axis=1, keepdims=True)
        sel = jnp.min(jnp.where(d2 == mn, col, jnp.int32(1 << 30)),
                      axis=1, keepdims=True)
        onehot = col == sel
        d2 = jnp.where(onehot, jnp.inf, d2)
        w = 1.0 / jnp.maximum(mn, 1e-16)
        wmat = wmat + jnp.where(onehot, w, 0.0)
        wsum = wsum + w
    interp = jnp.dot(wmat, hsrc_ref[...], preferred_element_type=_F32) / wsum
    h = jnp.concatenate([interp, skip_ref[...]], axis=1)
    out_ref[...] = _mlp_val(h, wbs, plain_last=plain_last)


def _fp_stage(posq, pos_src, h_src, skip, nns, n_src, block_q,
              plain_last=False):
    mq_pad = posq.shape[0]
    block_q = min(block_q, mq_pad)
    np_pad = pos_src.shape[0]
    posq8 = _pad2(posq, mq_pad, 8)
    post = _pad2(pos_src.T, 8, np_pad)
    wbs = [wb for nn in nns for wb in nn]
    args = [posq8, post, h_src, skip]
    full = lambda s: pl.BlockSpec(s, lambda i: (0, 0))
    specs = [
        pl.BlockSpec((block_q, 8), lambda i: (i, 0)),
        full((8, np_pad)),
        full(h_src.shape),
        pl.BlockSpec((block_q, skip.shape[1]), lambda i: (i, 0)),
    ]
    for (w, b) in wbs:
        args += [w, b.reshape(1, -1)]
        specs += [full(w.shape), full((1, w.shape[1]))]
    h_out = wbs[-1][0].shape[1]
    grid = mq_pad // block_q
    kern = functools.partial(_fp_kernel, n_src=n_src, k_nn=3,
                             plain_last=plain_last)
    return pl.pallas_call(
        kern,
        grid=(grid,),
        in_specs=specs,
        out_specs=pl.BlockSpec((block_q, h_out), lambda i: (i, 0)),
        out_shape=jax.ShapeDtypeStruct((mq_pad, h_out), _F32),
    )(*args)


def kernel(x, pos, batch, sa1_nn, sa2_nn, gsa_nn, fp1_nn, fp2_nn, fp3_nn,
           head_nn):
    del batch
    n = pos.shape[0]
    m1 = int(math.ceil(0.2 * n))
    m2 = int(math.ceil(0.25 * m1))
    m1p = ((m1 + 127) // 128) * 128
    m2p = ((m2 + 127) // 128) * 128

    posq1 = _fps(pos, n, m1)
    posq1p = _pad2(posq1, m1p, 3)
    x1 = _sa_conv(posq1p, pos, x, sa1_nn, n, 128)

    pos1p = posq1p
    posq2 = _fps(pos1p, m1, m2)
    posq2p = _pad2(posq2, m2p, 3)
    x1z = _pad2(x1[:m1], m1p, x1.shape[1])
    x2 = _sa_conv(posq2p, pos1p, x1z, sa2_nn, m1, 128)

    r2 = ((m2 + 7) // 8) * 8
    x2c = _pad2(x2[:m2], r2, x2.shape[1])
    pos2c = _pad2(posq2, r2, 3)
    h1 = _gsa_fp1(x2c, pos2c, gsa_nn, fp1_nn, m2)

    np2 = ((m2 + 127) // 128) * 128
    h1p = _pad2(h1[:m2], np2, h1.shape[1])
    pos2p = _pad2(posq2, np2, 3)
    h2 = _fp_stage(posq1p, pos2p, h1p, x1z, [fp2_nn], m2, 832)

    h2p = _pad2(h2[:m1], m1p, h2.shape[1])
    out = _fp_stage(pos, pos1p, h2p, x, [fp3_nn, head_nn], m1, 512,
                    plain_last=True)
    return out[:n]


# speedup vs baseline: 1.0635x; 1.0635x over previous
import functools
import math

import jax
import jax.numpy as jnp
from jax.experimental import pallas as pl
from jax.experimental.pallas import tpu as pltpu

_NUM_CONN = 16
_R2 = 1000.0 * 1000.0
_F32 = jnp.float32


def _pad2(a, rows, cols):
    return jnp.pad(a, ((0, rows - a.shape[0]), (0, cols - a.shape[1])))


def _mlp_val(h, wbs, plain_last=False):
    n = len(wbs)
    for i, (w_ref, b_ref) in enumerate(wbs):
        h = jnp.dot(h, w_ref[...], preferred_element_type=_F32) + b_ref[...]
        if not (plain_last and i == n - 1):
            h = jnp.maximum(h, 0.0)
    return h


def _fps_kernel(px_ref, py_ref, pz_ref, idx_ref, qx_ref, qy_ref, qz_ref,
                *, n_real, m):
    px = px_ref[...]
    py = py_ref[...]
    pz = pz_ref[...]
    shp = px.shape
    gidx = (jax.lax.broadcasted_iota(jnp.int32, shp, 0) * shp[1]
            + jax.lax.broadcasted_iota(jnp.int32, shp, 1))
    dists0 = jnp.where(gidx < n_real, jnp.inf, -1.0).astype(_F32)

    idx_ref[pl.ds(0, 1), :] = jnp.zeros((1, 1), jnp.int32)
    l0x = px[0, 0]
    l0y = py[0, 0]
    l0z = pz[0, 0]
    qx_ref[pl.ds(0, 1), :] = jnp.reshape(l0x, (1, 1))
    qy_ref[pl.ds(0, 1), :] = jnp.reshape(l0y, (1, 1))
    qz_ref[pl.ds(0, 1), :] = jnp.reshape(l0z, (1, 1))

    def body(i, carry):
        dists, lx, ly, lz = carry
        dx = px - lx
        dy = py - ly
        dz = pz - lz
        d = dx * dx + dy * dy + dz * dz
        dists = jnp.minimum(dists, d)
        mx = jnp.max(dists)
        j = jnp.min(jnp.where(dists == mx, gidx, jnp.int32(1 << 30)))
        onehot = gidx == j
        nlx = jnp.sum(jnp.where(onehot, px, 0.0))
        nly = jnp.sum(jnp.where(onehot, py, 0.0))
        nlz = jnp.sum(jnp.where(onehot, pz, 0.0))
        idx_ref[pl.ds(i, 1), :] = jnp.reshape(j, (1, 1))
        qx_ref[pl.ds(i, 1), :] = jnp.reshape(nlx, (1, 1))
        qy_ref[pl.ds(i, 1), :] = jnp.reshape(nly, (1, 1))
        qz_ref[pl.ds(i, 1), :] = jnp.reshape(nlz, (1, 1))
        return dists, nlx, nly, nlz

    jax.lax.fori_loop(1, m, body, (dists0, l0x, l0y, l0z))


def _fps(pos, n_real, m):
    n_pad = pos.shape[0]
    rows = n_pad // 128
    px = pos[:, 0].reshape(rows, 128)
    py = pos[:, 1].reshape(rows, 128)
    pz = pos[:, 2].reshape(rows, 128)
    m_pad = ((m + 7) // 8) * 8
    out = pl.pallas_call(
        functools.partial(_fps_kernel, n_real=n_real, m=m),
        out_shape=[
            jax.ShapeDtypeStruct((m_pad, 1), jnp.int32),
            jax.ShapeDtypeStruct((m_pad, 1), _F32),
            jax.ShapeDtypeStruct((m_pad, 1), _F32),
            jax.ShapeDtypeStruct((m_pad, 1), _F32),
        ],
    )(px, py, pz)
    _, qx, qy, qz = out
    return jnp.concatenate([qx, qy, qz], axis=1)[:m]


def _sa_kernel(posq_ref, post_ref, g_ref, w1, b1, w2, b2, w3, b3,
               out_ref, e_scr, v_scr, oh_scr, *, n_cand, c_feat, k_nn):
    posq = posq_ref[...]
    post = post_ref[...]
    bq = posq.shape[0]
    np_ = post.shape[1]
    aa = jnp.sum(posq * posq, axis=1, keepdims=True)
    bb = jnp.sum(post * post, axis=0, keepdims=True)
    ab = jnp.dot(posq, post, preferred_element_type=_F32)
    d2 = jnp.maximum(aa + bb - 2.0 * ab, 0.0)
    col = jax.lax.broadcasted_iota(jnp.int32, (bq, np_), 1)
    d2 = jnp.where((col < n_cand) & (d2 <= _R2), d2, jnp.inf)
    g = g_ref[...]

    for k in range(k_nn):
        mn = jnp.min(d2, axis=1, keepdims=True)
        sel = jnp.min(jnp.where(d2 == mn, col, jnp.int32(1 << 30)),
                      axis=1, keepdims=True)
        onehot = col == sel
        d2 = jnp.where(onehot, jnp.inf, d2)
        oh_scr[pl.ds((k % 2) * bq, bq), :] = onehot.astype(_F32)
        if k % 2 == 1:
            e2 = jnp.dot(oh_scr[...], g, preferred_element_type=_F32)
            e_scr[pl.ds((k - 1) * bq, 2 * bq), :] = e2
        v_scr[pl.ds(k * bq, bq), :] = (mn < jnp.inf).astype(_F32)

    cg = g.shape[1]
    e_all = e_scr[...]
    posq3 = posq[:, :3]
    corr = jnp.concatenate(
        [jnp.zeros((k_nn * bq, c_feat), _F32),
         jnp.tile(posq3, (k_nn, 1)),
         jnp.zeros((k_nn * bq, cg - c_feat - 3), _F32)], axis=1)
    h = _mlp_val(e_all - corr, [(w1, b1), (w2, b2), (w3, b3)])
    h = jnp.where(v_scr[...] > 0.0, h, -jnp.inf)
    h = jnp.max(jnp.reshape(h, (k_nn, bq, h.shape[1])), axis=0)
    out_ref[...] = h


def _sa_conv(posq, pos_cand, feats, nn, n_cand, block_q):
    mq_pad = posq.shape[0]
    block_q = min(block_q, mq_pad)
    np_pad = pos_cand.shape[0]
    c_feat = feats.shape[1]
    cg = ((c_feat + 3 + 7) // 8) * 8
    g = _pad2(jnp.concatenate([feats, pos_cand], axis=1), np_pad, cg)
    posq8 = _pad2(posq, mq_pad, 8)
    post = _pad2(pos_cand.T, 8, np_pad)
    (w1, bb1), (w2, bb2), (w3, bb3) = nn
    w1p = _pad2(w1, cg, w1.shape[1])
    h3 = w3.shape[1]
    grid = mq_pad // block_q
    kern = functools.partial(_sa_kernel, n_cand=n_cand, c_feat=c_feat,
                             k_nn=_NUM_CONN)
    full = lambda s: pl.BlockSpec(s, lambda i: (0, 0))
    return pl.pallas_call(
        kern,
        grid=(grid,),
        in_specs=[
            pl.BlockSpec((block_q, 8), lambda i: (i, 0)),
            full((8, np_pad)),
            full((np_pad, cg)),
            full(w1p.shape), full((1, w1.shape[1])),
            full(w2.shape), full((1, w2.shape[1])),
            full(w3.shape), full((1, h3)),
        ],
        out_specs=pl.BlockSpec((block_q, h3), lambda i: (i, 0)),
        out_shape=jax.ShapeDtypeStruct((mq_pad, h3), _F32),
        scratch_shapes=[
            pltpu.VMEM((_NUM_CONN * block_q, cg), _F32),
            pltpu.VMEM((_NUM_CONN * block_q, 1), _F32),
            pltpu.VMEM((2 * block_q, np_pad), _F32),
        ],
    )(posq8, post, g, w1p, bb1.reshape(1, -1), w2, bb2.reshape(1, -1),
      w3, bb3.reshape(1, -1))


def _gsa_kernel(x2_ref, pos2_ref, gw1, gb1, gw2, gb2, gw3, gb3,
                fw1, fb1, fw2, fb2, fw3, fb3, out_ref, *, n_real):
    x2 = x2_ref[...]
    pos2 = pos2_ref[...][:, :3]
    h = jnp.concatenate([x2, pos2], axis=1)
    hg = _mlp_val(h, [(gw1, gb1), (gw2, gb2), (gw3, gb3)])
    row = jax.lax.broadcasted_iota(jnp.int32, hg.shape, 0)
    hg = jnp.where(row < n_real, hg, -jnp.inf)
    xg = jnp.max(hg, axis=0, keepdims=True)
    xg_b = jnp.broadcast_to(xg, (x2.shape[0], xg.shape[1]))
    fin = jnp.concatenate([xg_b, x2], axis=1)
    out_ref[...] = _mlp_val(fin, [(fw1, fb1), (fw2, fb2), (fw3, fb3)])


def _gsa_fp1(x2, pos2, gsa_nn, fp1_nn, n_real):
    rows = x2.shape[0]
    args = [x2, _pad2(pos2, rows, 8)]
    for (w, b) in list(gsa_nn) + list(fp1_nn):
        args += [w, b.reshape(1, -1)]
    h_out = fp1_nn[-1][0].shape[1]
    return pl.pallas_call(
        functools.partial(_gsa_kernel, n_real=n_real),
        out_shape=jax.ShapeDtypeStruct((rows, h_out), _F32),
    )(*args)


def _fp_kernel(posq_ref, post_ref, hsrc_ref, skip_ref,
               *mlp_refs, n_src, k_nn, plain_last):
    out_ref = mlp_refs[-1]
    wbs = [(mlp_refs[2 * i], mlp_refs[2 * i + 1])
           for i in range((len(mlp_refs) - 1) // 2)]
    posq = posq_ref[...]
    post = post_ref[...]
    bq = posq.shape[0]
    np_ = post.shape[1]
    aa = jnp.sum(posq * posq, axis=1, keepdims=True)
    bb = jnp.sum(post * post, axis=0, keepdims=True)
    ab = jnp.dot(posq, post, preferred_element_type=_F32)
    d2 = jnp.maximum(aa + bb - 2.0 * ab, 0.0)
    col = jax.lax.broadcasted_iota(jnp.int32, (bq, np_), 1)
    d2 = jnp.where(col < n_src, d2, jnp.inf)

    wmat = jnp.zeros((bq, np_), _F32)
    wsum = jnp.zeros((bq, 1), _F32)
    for _ in range(k_nn):
        mn = jnp.min(d2, axis=1, keepdims=True)
        sel = jnp.min(jnp.where(d2 == mn, col, jnp.int32(1 << 30)),
                      axis=1, keepdims=True)
        onehot = col == sel
        d2 = jnp.where(onehot, jnp.inf, d2)
        w = 1.0 / jnp.maximum(mn, 1e-16)
        wmat = wmat + jnp.where(onehot, w, 0.0)
        wsum = wsum + w
    interp = jnp.dot(wmat, hsrc_ref[...], preferred_element_type=_F32) / wsum
    h = jnp.concatenate([interp, skip_ref[...]], axis=1)
    out_ref[...] = _mlp_val(h, wbs, plain_last=plain_last)


def _fp_stage(posq, pos_src, h_src, skip, nns, n_src, block_q,
              plain_last=False):
    mq_pad = posq.shape[0]
    block_q = min(block_q, mq_pad)
    np_pad = pos_src.shape[0]
    posq8 = _pad2(posq, mq_pad, 8)
    post = _pad2(pos_src.T, 8, np_pad)
    wbs = [wb for nn in nns for wb in nn]
    args = [posq8, post, h_src, skip]
    full = lambda s: pl.BlockSpec(s, lambda i: (0, 0))
    specs = [
        pl.BlockSpec((block_q, 8), lambda i: (i, 0)),
        full((8, np_pad)),
        full(h_src.shape),
        pl.BlockSpec((block_q, skip.shape[1]), lambda i: (i, 0)),
    ]
    for (w, b) in wbs:
        args += [w, b.reshape(1, -1)]
        specs += [full(w.shape), full((1, w.shape[1]))]
    h_out = wbs[-1][0].shape[1]
    grid = mq_pad // block_q
    kern = functools.partial(_fp_kernel, n_src=n_src, k_nn=3,
                             plain_last=plain_last)
    return pl.pallas_call(
        kern,
        grid=(grid,),
        in_specs=specs,
        out_specs=pl.BlockSpec((block_q, h_out), lambda i: (i, 0)),
        out_shape=jax.ShapeDtypeStruct((mq_pad, h_out), _F32),
    )(*args)


def kernel(x, pos, batch, sa1_nn, sa2_nn, gsa_nn, fp1_nn, fp2_nn, fp3_nn,
           head_nn):
    del batch
    n = pos.shape[0]
    m1 = int(math.ceil(0.2 * n))
    m2 = int(math.ceil(0.25 * m1))
    m1p = ((m1 + 127) // 128) * 128
    m2p = ((m2 + 127) // 128) * 128

    posq1 = _fps(pos, n, m1)
    posq1p = _pad2(posq1, m1p, 3)
    x1 = _sa_conv(posq1p, pos, x, sa1_nn, n, 64)

    pos1p = posq1p
    posq2 = _fps(pos1p, m1, m2)
    posq2p = _pad2(posq2, m2p, 3)
    x1z = _pad2(x1[:m1], m1p, x1.shape[1])
    x2 = _sa_conv(posq2p, pos1p, x1z, sa2_nn, m1, 128)

    r2 = ((m2 + 7) // 8) * 8
    x2c = _pad2(x2[:m2], r2, x2.shape[1])
    pos2c = _pad2(posq2, r2, 3)
    h1 = _gsa_fp1(x2c, pos2c, gsa_nn, fp1_nn, m2)

    np2 = ((m2 + 127) // 128) * 128
    h1p = _pad2(h1[:m2], np2, h1.shape[1])
    pos2p = _pad2(posq2, np2, 3)
    h2 = _fp_stage(posq1p, pos2p, h1p, x1z, [fp2_nn], m2, 832)

    h2p = _pad2(h2[:m1], m1p, h2.shape[1])
    out = _fp_stage(pos, pos1p, h2p, x, [fp3_nn, head_nn], m1, 256,
                    plain_last=True)
    return out[:n]


# speedup vs baseline: 1.1010x; 1.0353x over previous
import functools
import math

import jax
import jax.numpy as jnp
from jax.experimental import pallas as pl
from jax.experimental.pallas import tpu as pltpu

_NUM_CONN = 16
_R2 = 1000.0 * 1000.0
_F32 = jnp.float32


def _pad2(a, rows, cols):
    return jnp.pad(a, ((0, rows - a.shape[0]), (0, cols - a.shape[1])))


def _mlp_val(h, wbs, plain_last=False):
    n = len(wbs)
    for i, (w_ref, b_ref) in enumerate(wbs):
        h = jnp.dot(h, w_ref[...], preferred_element_type=_F32) + b_ref[...]
        if not (plain_last and i == n - 1):
            h = jnp.maximum(h, 0.0)
    return h


def _fps_kernel(px_ref, py_ref, pz_ref, prow_ref, idx_ref, qx_ref, qy_ref,
                qz_ref, *, n_real, m):
    px = px_ref[...]
    py = py_ref[...]
    pz = pz_ref[...]
    shp = px.shape
    gidx = (jax.lax.broadcasted_iota(jnp.int32, shp, 0) * shp[1]
            + jax.lax.broadcasted_iota(jnp.int32, shp, 1))
    dists0 = jnp.where(gidx < n_real, jnp.inf, -1.0).astype(_F32)

    idx_ref[pl.ds(0, 1), :] = jnp.zeros((1, 1), jnp.int32)
    l0x = px[0, 0]
    l0y = py[0, 0]
    l0z = pz[0, 0]
    qx_ref[pl.ds(0, 1), :] = jnp.reshape(l0x, (1, 1))
    qy_ref[pl.ds(0, 1), :] = jnp.reshape(l0y, (1, 1))
    qz_ref[pl.ds(0, 1), :] = jnp.reshape(l0z, (1, 1))

    def body(i, carry):
        dists, lx, ly, lz = carry
        dx = px - lx
        dy = py - ly
        dz = pz - lz
        d = dx * dx + dy * dy + dz * dz
        dists = jnp.minimum(dists, d)
        mx = jnp.max(dists)
        j = jnp.min(jnp.where(dists == mx, gidx, jnp.int32(1 << 30)))
        row = prow_ref[pl.ds(j, 1), :]
        nlx = row[0, 0]
        nly = row[0, 1]
        nlz = row[0, 2]
        idx_ref[pl.ds(i, 1), :] = jnp.reshape(j, (1, 1))
        qx_ref[pl.ds(i, 1), :] = jnp.reshape(nlx, (1, 1))
        qy_ref[pl.ds(i, 1), :] = jnp.reshape(nly, (1, 1))
        qz_ref[pl.ds(i, 1), :] = jnp.reshape(nlz, (1, 1))
        return dists, nlx, nly, nlz

    jax.lax.fori_loop(1, m, body, (dists0, l0x, l0y, l0z))


def _fps(pos, n_real, m):
    n_pad = pos.shape[0]
    rows = n_pad // 128
    px = pos[:, 0].reshape(rows, 128)
    py = pos[:, 1].reshape(rows, 128)
    pz = pos[:, 2].reshape(rows, 128)
    prow = _pad2(pos, n_pad, 4)
    m_pad = ((m + 7) // 8) * 8
    out = pl.pallas_call(
        functools.partial(_fps_kernel, n_real=n_real, m=m),
        out_shape=[
            jax.ShapeDtypeStruct((m_pad, 1), jnp.int32),
            jax.ShapeDtypeStruct((m_pad, 1), _F32),
            jax.ShapeDtypeStruct((m_pad, 1), _F32),
            jax.ShapeDtypeStruct((m_pad, 1), _F32),
        ],
    )(px, py, pz, prow)
    _, qx, qy, qz = out
    return jnp.concatenate([qx, qy, qz], axis=1)[:m]


def _sa_kernel(posq_ref, post_ref, g_ref, w1, b1, w2, b2, w3, b3,
               out_ref, e_scr, v_scr, *, n_cand, c_feat, k_nn):
    posq = posq_ref[...]
    post = post_ref[...]
    bq = posq.shape[0]
    np_ = post.shape[1]
    aa = jnp.sum(posq * posq, axis=1, keepdims=True)
    bb = jnp.sum(post * post, axis=0, keepdims=True)
    ab = jnp.dot(posq, post, preferred_element_type=_F32)
    d2 = jnp.maximum(aa + bb - 2.0 * ab, 0.0)
    col = jax.lax.broadcasted_iota(jnp.int32, (bq, np_), 1)
    d2 = jnp.where((col < n_cand) & (d2 <= _R2), d2, jnp.inf)
    g = g_ref[...]

    for k in range(k_nn):
        mn = jnp.min(d2, axis=1, keepdims=True)
        sel = jnp.min(jnp.where(d2 == mn, col, jnp.int32(1 << 30)),
                      axis=1, keepdims=True)
        onehot = col == sel
        d2 = jnp.where(onehot, jnp.inf, d2)
        e = jnp.dot(onehot.astype(_F32), g, preferred_element_type=_F32)
        e_scr[pl.ds(k * bq, bq), :] = e
        v_scr[pl.ds(k * bq, bq), :] = (mn < jnp.inf).astype(_F32)

    cg = g.shape[1]
    e_all = e_scr[...]
    posq3 = posq[:, :3]
    corr = jnp.concatenate(
        [jnp.zeros((k_nn * bq, c_feat), _F32),
         jnp.tile(posq3, (k_nn, 1)),
         jnp.zeros((k_nn * bq, cg - c_feat - 3), _F32)], axis=1)
    h = _mlp_val(e_all - corr, [(w1, b1), (w2, b2), (w3, b3)])
    h = jnp.where(v_scr[...] > 0.0, h, -jnp.inf)
    h = jnp.max(jnp.reshape(h, (k_nn, bq, h.shape[1])), axis=0)
    out_ref[...] = h


def _sa_conv(posq, pos_cand, feats, nn, n_cand, block_q):
    mq_pad = posq.shape[0]
    block_q = min(block_q, mq_pad)
    np_pad = pos_cand.shape[0]
    c_feat = feats.shape[1]
    cg = ((c_feat + 3 + 7) // 8) * 8
    g = _pad2(jnp.concatenate([feats, pos_cand], axis=1), np_pad, cg)
    posq8 = _pad2(posq, mq_pad, 8)
    post = _pad2(pos_cand.T, 8, np_pad)
    (w1, bb1), (w2, bb2), (w3, bb3) = nn
    w1p = _pad2(w1, cg, w1.shape[1])
    h3 = w3.shape[1]
    grid = mq_pad // block_q
    kern = functools.partial(_sa_kernel, n_cand=n_cand, c_feat=c_feat,
                             k_nn=_NUM_CONN)
    full = lambda s: pl.BlockSpec(s, lambda i: (0, 0))
    return pl.pallas_call(
        kern,
        grid=(grid,),
        in_specs=[
            pl.BlockSpec((block_q, 8), lambda i: (i, 0)),
            full((8, np_pad)),
            full((np_pad, cg)),
            full(w1p.shape), full((1, w1.shape[1])),
            full(w2.shape), full((1, w2.shape[1])),
            full(w3.shape), full((1, h3)),
        ],
        out_specs=pl.BlockSpec((block_q, h3), lambda i: (i, 0)),
        out_shape=jax.ShapeDtypeStruct((mq_pad, h3), _F32),
        scratch_shapes=[
            pltpu.VMEM((_NUM_CONN * block_q, cg), _F32),
            pltpu.VMEM((_NUM_CONN * block_q, 1), _F32),
        ],
    )(posq8, post, g, w1p, bb1.reshape(1, -1), w2, bb2.reshape(1, -1),
      w3, bb3.reshape(1, -1))


def _gsa_kernel(x2_ref, pos2_ref, gw1, gb1, gw2, gb2, gw3, gb3,
                fw1, fb1, fw2, fb2, fw3, fb3, out_ref, *, n_real):
    x2 = x2_ref[...]
    pos2 = pos2_ref[...][:, :3]
    h = jnp.concatenate([x2, pos2], axis=1)
    hg = _mlp_val(h, [(gw1, gb1), (gw2, gb2), (gw3, gb3)])
    row = jax.lax.broadcasted_iota(jnp.int32, hg.shape, 0)
    hg = jnp.where(row < n_real, hg, -jnp.inf)
    xg = jnp.max(hg, axis=0, keepdims=True)
    xg_b = jnp.broadcast_to(xg, (x2.shape[0], xg.shape[1]))
    fin = jnp.concatenate([xg_b, x2], axis=1)
    out_ref[...] = _mlp_val(fin, [(fw1, fb1), (fw2, fb2), (fw3, fb3)])


def _gsa_fp1(x2, pos2, gsa_nn, fp1_nn, n_real):
    rows = x2.shape[0]
    args = [x2, _pad2(pos2, rows, 8)]
    for (w, b) in list(gsa_nn) + list(fp1_nn):
        args += [w, b.reshape(1, -1)]
    h_out = fp1_nn[-1][0].shape[1]
    return pl.pallas_call(
        functools.partial(_gsa_kernel, n_real=n_real),
        out_shape=jax.ShapeDtypeStruct((rows, h_out), _F32),
    )(*args)


def _fp_kernel(posq_ref, post_ref, hsrc_ref, skip_ref,
               *mlp_refs, n_src, k_nn, plain_last):
    out_ref = mlp_refs[-1]
    wbs = [(mlp_refs[2 * i], mlp_refs[2 * i + 1])
           for i in range((len(mlp_refs) - 1) // 2)]
    posq = posq_ref[...]
    post = post_ref[...]
    bq = posq.shape[0]
    np_ = post.shape[1]
    aa = jnp.sum(posq * posq, axis=1, keepdims=True)
    bb = jnp.sum(post * post, axis=0, keepdims=True)
    ab = jnp.dot(posq, post, preferred_element_type=_F32)
    d2 = jnp.maximum(aa + bb - 2.0 * ab, 0.0)
    col = jax.lax.broadcasted_iota(jnp.int32, (bq, np_), 1)
    d2 = jnp.where(col < n_src, d2, jnp.inf)

    wmat = jnp.zeros((bq, np_), _F32)
    wsum = jnp.zeros((bq, 1), _F32)
    for _ in range(k_nn):
        mn = jnp.min(d2, axis=1, keepdims=True)
        sel = jnp.min(jnp.where(d2 == mn, col, jnp.int32(1 << 30)),
                      axis=1, keepdims=True)
        onehot = col == sel
        d2 = jnp.where(onehot, jnp.inf, d2)
        w = 1.0 / jnp.maximum(mn, 1e-16)
        wmat = wmat + jnp.where(onehot, w, 0.0)
        wsum = wsum + w
    interp = jnp.dot(wmat, hsrc_ref[...], preferred_element_type=_F32) / wsum
    h = jnp.concatenate([interp, skip_ref[...]], axis=1)
    out_ref[...] = _mlp_val(h, wbs, plain_last=plain_last)


def _fp_stage(posq, pos_src, h_src, skip, nns, n_src, block_q,
              plain_last=False):
    mq_pad = posq.shape[0]
    block_q = min(block_q, mq_pad)
    np_pad = pos_src.shape[0]
    posq8 = _pad2(posq, mq_pad, 8)
    post = _pad2(pos_src.T, 8, np_pad)
    wbs = [wb for nn in nns for wb in nn]
    args = [posq8, post, h_src, skip]
    full = lambda s: pl.BlockSpec(s, lambda i: (0, 0))
    specs = [
        pl.BlockSpec((block_q, 8), lambda i: (i, 0)),
        full((8, np_pad)),
        full(h_src.shape),
        pl.BlockSpec((block_q, skip.shape[1]), lambda i: (i, 0)),
    ]
    for (w, b) in wbs:
        args += [w, b.reshape(1, -1)]
        specs += [full(w.shape), full((1, w.shape[1]))]
    h_out = wbs[-1][0].shape[1]
    grid = mq_pad // block_q
    kern = functools.partial(_fp_kernel, n_src=n_src, k_nn=3,
                             plain_last=plain_last)
    return pl.pallas_call(
        kern,
        grid=(grid,),
        in_specs=specs,
        out_specs=pl.BlockSpec((block_q, h_out), lambda i: (i, 0)),
        out_shape=jax.ShapeDtypeStruct((mq_pad, h_out), _F32),
    )(*args)


def kernel(x, pos, batch, sa1_nn, sa2_nn, gsa_nn, fp1_nn, fp2_nn, fp3_nn,
           head_nn):
    del batch
    n = pos.shape[0]
    m1 = int(math.ceil(0.2 * n))
    m2 = int(math.ceil(0.25 * m1))
    m1p = ((m1 + 127) // 128) * 128
    m2p = ((m2 + 127) // 128) * 128

    posq1 = _fps(pos, n, m1)
    posq1p = _pad2(posq1, m1p, 3)
    x1 = _sa_conv(posq1p, pos, x, sa1_nn, n, 64)

    pos1p = posq1p
    posq2 = _fps(pos1p, m1, m2)
    posq2p = _pad2(posq2, m2p, 3)
    x1z = _pad2(x1[:m1], m1p, x1.shape[1])
    x2 = _sa_conv(posq2p, pos1p, x1z, sa2_nn, m1, 128)

    r2 = ((m2 + 7) // 8) * 8
    x2c = _pad2(x2[:m2], r2, x2.shape[1])
    pos2c = _pad2(posq2, r2, 3)
    h1 = _gsa_fp1(x2c, pos2c, gsa_nn, fp1_nn, m2)

    np2 = ((m2 + 127) // 128) * 128
    h1p = _pad2(h1[:m2], np2, h1.shape[1])
    pos2p = _pad2(posq2, np2, 3)
    h2 = _fp_stage(posq1p, pos2p, h1p, x1z, [fp2_nn], m2, 832)

    h2p = _pad2(h2[:m1], m1p, h2.shape[1])
    out = _fp_stage(pos, pos1p, h2p, x, [fp3_nn, head_nn], m1, 256,
                    plain_last=True)
    return out[:n]
